# baseline (speedup 1.0000x reference)
import functools

import jax
import jax.numpy as jnp
from jax import lax
from jax.experimental import pallas as pl
from jax.experimental.pallas import tpu as pltpu
from jax.experimental.pallas import tpu_sc as plsc

_VOCAB = 1000000
_EMBED_DIM = 16
_BATCH = 16384

_NC = 2
_NS = 16
_NW = _NC * _NS
_B_PER_W = _BATCH // _NW
_CHUNK = 128
_NCHUNK = _B_PER_W // _CHUNK

_mesh = plsc.VectorSubcoreMesh(core_axis_name="c", subcore_axis_name="s")


@functools.partial(
    pl.kernel,
    mesh=_mesh,
    out_type=jax.ShapeDtypeStruct((_BATCH, _EMBED_DIM), jnp.float32),
    scratch_types=[
        pltpu.VMEM((_NCHUNK, _CHUNK), jnp.int32),
        pltpu.VMEM((_B_PER_W, _EMBED_DIM), jnp.float32),
        pltpu.SemaphoreType.DMA,
    ],
    compiler_params=pltpu.CompilerParams(use_tc_tiling_on_sc=False),
)
def _gather_kernel(table_hbm, idx_hbm, out_hbm, idx_v, rows_v, sem):
    wid = lax.axis_index("s") * _NC + lax.axis_index("c")
    base = wid * _B_PER_W
    pltpu.sync_copy(idx_hbm.at[pl.ds(wid * _NCHUNK, _NCHUNK)], idx_v)
    copies = [
        pltpu.async_copy(
            table_hbm.at[idx_v.at[j]],
            rows_v.at[pl.ds(j * _CHUNK, _CHUNK)],
            sem,
        )
        for j in range(_NCHUNK)
    ]
    for c in copies:
        c.wait()
    pltpu.sync_copy(rows_v, out_hbm.at[pl.ds(base, _B_PER_W)])


def kernel(data, channelindex):
    idx2d = channelindex.astype(jnp.int32).reshape(_BATCH // _CHUNK, _CHUNK)
    return _gather_kernel(data, idx2d)


# speedup vs baseline: 5.3000x; 5.3000x over previous
import functools

import jax
import jax.numpy as jnp
from jax import lax
from jax.experimental import pallas as pl
from jax.experimental.pallas import tpu as pltpu
from jax.experimental.pallas import tpu_sc as plsc

_VOCAB = 1000000
_D = 16
_B = 16384

_NC = 2
_NS = 16
_NW = _NC * _NS
_BPW = _B // _NW
_G = 16
_NG = _BPW // _G

_mesh = plsc.VectorSubcoreMesh(core_axis_name="c", subcore_axis_name="s")


@functools.partial(
    pl.kernel,
    mesh=_mesh,
    out_type=jax.ShapeDtypeStruct((_D, _B), jnp.float32),
    scratch_types=[
        pltpu.VMEM((_BPW,), jnp.int32),
        pltpu.VMEM((_G, _D, 128), jnp.float32),
        pltpu.VMEM((_D, _BPW), jnp.float32),
        pltpu.SemaphoreType.DMA,
    ],
    compiler_params=pltpu.CompilerParams(needs_layout_passes=False),
)
def _gather_kernel(table_t, idx_hbm, out_t, idx_v, blocks, outbuf, sem):
    wid = lax.axis_index("s") * _NC + lax.axis_index("c")
    base = wid * _BPW
    pltpu.sync_copy(idx_hbm.at[pl.ds(base, _BPW)], idx_v)
    lanes = lax.iota(jnp.int32, _D)

    def group(g, carry):
        gbase = g * _G
        ivec = idx_v[pl.ds(gbase, _G)]
        cvec = (ivec // 128) * 128
        copies = []
        for j in range(_G):
            cj = jnp.sum(jnp.where(lanes == j, cvec, 0))
            c0 = pl.multiple_of(cj, 128)
            copies.append(
                pltpu.async_copy(table_t.at[:, pl.ds(c0, 128)], blocks.at[j], sem)
            )
        for cp in copies:
            cp.wait()
        ovec = jnp.bitwise_and(ivec, 127)
        for d in range(_D):
            col = plsc.load_gather(
                blocks, [lanes, jnp.full((_G,), d, jnp.int32), ovec]
            )
            outbuf[d, pl.ds(gbase, _G)] = col
        return carry

    lax.fori_loop(0, _NG, group, 0)
    pltpu.sync_copy(outbuf, out_t.at[:, pl.ds(base, _BPW)])


def kernel(data, channelindex):
    out_t = _gather_kernel(data.T, channelindex.astype(jnp.int32))
    return out_t.T


# speedup vs baseline: 5.9615x; 1.1248x over previous
import functools

import jax
import jax.numpy as jnp
from jax import lax
from jax.experimental import pallas as pl
from jax.experimental.pallas import tpu as pltpu
from jax.experimental.pallas import tpu_sc as plsc

_VOCAB = 1000000
_D = 16
_B = 16384

_NC = 2
_NS = 16
_NW = _NC * _NS
_BPW = _B // _NW
_G = 32
_NG = _BPW // _G

_mesh = plsc.VectorSubcoreMesh(core_axis_name="c", subcore_axis_name="s")


@functools.partial(
    pl.kernel,
    mesh=_mesh,
    out_type=jax.ShapeDtypeStruct((_D, _B), jnp.float32),
    scratch_types=[
        pltpu.VMEM((_BPW,), jnp.int32),
        pltpu.VMEM((_G, _D, 128), jnp.float32),
        pltpu.VMEM((_D, _BPW), jnp.float32),
        pltpu.SemaphoreType.DMA,
    ],
    compiler_params=pltpu.CompilerParams(needs_layout_passes=False),
)
def _gather_kernel(table_t, idx_hbm, out_t, idx_v, blocks, outbuf, sem):
    wid = lax.axis_index("s") * _NC + lax.axis_index("c")
    base = wid * _BPW
    pltpu.sync_copy(idx_hbm.at[pl.ds(base, _BPW)], idx_v)
    lanes = lax.iota(jnp.int32, _D)

    def group(g, carry):
        gbase = g * _G
        copies = []
        for h in range(_G // 16):
            ivec = idx_v[pl.ds(gbase + h * 16, 16)]
            cvec = (ivec // 128) * 128
            for j in range(16):
                cj = jnp.sum(jnp.where(lanes == j, cvec, 0))
                c0 = pl.multiple_of(cj, 128)
                copies.append(
                    pltpu.async_copy(
                        table_t.at[:, pl.ds(c0, 128)], blocks.at[h * 16 + j], sem
                    )
                )
        for cp in copies:
            cp.wait()
        for h in range(_G // 16):
            ivec = idx_v[pl.ds(gbase + h * 16, 16)]
            ovec = jnp.bitwise_and(ivec, 127)
            for d in range(_D):
                col = plsc.load_gather(
                    blocks, [lanes + h * 16, jnp.full((16,), d, jnp.int32), ovec]
                )
                outbuf[d, pl.ds(gbase + h * 16, 16)] = col
        return carry

    lax.fori_loop(0, _NG, group, 0)
    pltpu.sync_copy(outbuf, out_t.at[:, pl.ds(base, _BPW)])


def kernel(data, channelindex):
    out_t = _gather_kernel(data.T, channelindex.astype(jnp.int32))
    return out_t.T


# speedup vs baseline: 6.3386x; 1.0633x over previous
import functools

import jax
import jax.numpy as jnp
from jax import lax
from jax.experimental import pallas as pl
from jax.experimental.pallas import tpu as pltpu
from jax.experimental.pallas import tpu_sc as plsc

_VOCAB = 1000000
_D = 16
_B = 16384

_NC = 2
_NS = 16
_NW = _NC * _NS
_BPW = _B // _NW
_G = 16
_NG = _BPW // _G
_NSLOT = 3

_mesh = plsc.VectorSubcoreMesh(core_axis_name="c", subcore_axis_name="s")


@functools.partial(
    pl.kernel,
    mesh=_mesh,
    out_type=jax.ShapeDtypeStruct((_D, _B), jnp.float32),
    scratch_types=[
        pltpu.VMEM((_BPW,), jnp.int32),
        pltpu.VMEM((_NSLOT * _G, _D, 128), jnp.float32),
        pltpu.VMEM((_D, _BPW), jnp.float32),
        pltpu.SemaphoreType.DMA,
        pltpu.SemaphoreType.DMA,
        pltpu.SemaphoreType.DMA,
    ],
    compiler_params=pltpu.CompilerParams(needs_layout_passes=False),
)
def _gather_kernel(table_t, idx_hbm, out_t, idx_v, blocks, outbuf, s0, s1, s2):
    wid = lax.axis_index("s") * _NC + lax.axis_index("c")
    base = wid * _BPW
    pltpu.sync_copy(idx_hbm.at[pl.ds(base, _BPW)], idx_v)
    lanes = lax.iota(jnp.int32, _D)
    sems = (s0, s1, s2)

    def fire(g, slot):
        ivec = idx_v[pl.ds(g * _G, _G)]
        cvec = (ivec // 128) * 128
        copies = []
        for j in range(_G):
            cj = jnp.sum(jnp.where(lanes == j, cvec, 0))
            c0 = pl.multiple_of(cj, 128)
            copies.append(
                pltpu.async_copy(
                    table_t.at[:, pl.ds(c0, 128)],
                    blocks.at[slot * _G + j],
                    sems[slot],
                )
            )
        return copies

    def drain_extract(g, slot, copies):
        for cp in copies:
            cp.wait()
        ivec = idx_v[pl.ds(g * _G, _G)]
        ovec = jnp.bitwise_and(ivec, 127)
        blk = lanes + slot * _G
        for d in range(_D):
            col = plsc.load_gather(blocks, [blk, jnp.full((_G,), d, jnp.int32), ovec])
            outbuf[d, pl.ds(g * _G, _G)] = col

    pend0 = fire(0, 0)
    pend1 = fire(1, 1)

    def step(i, carry):
        g = 3 * i
        p2 = fire(g + 2, 2)
        drain_extract(g, 0, pend0)
        p0 = fire(g + 3, 0)
        drain_extract(g + 1, 1, pend1)
        p1 = fire(g + 4, 1)
        drain_extract(g + 2, 2, p2)
        return carry

    lax.fori_loop(0, 9, step, 0)
    g = 27
    p2 = fire(g + 2, 2)
    drain_extract(g, 0, pend0)
    p0 = fire(g + 3, 0)
    drain_extract(g + 1, 1, pend1)
    p1 = fire(g + 4, 1)
    drain_extract(g + 2, 2, p2)
    drain_extract(30, 0, p0)
    drain_extract(31, 1, p1)
    pltpu.sync_copy(outbuf, out_t.at[:, pl.ds(base, _BPW)])


def kernel(data, channelindex):
    out_t = _gather_kernel(data.T, channelindex.astype(jnp.int32))
    return out_t.T
